# baseline (speedup 1.0000x reference)
import functools

import jax
import jax.numpy as jnp
from jax import lax
from jax.experimental import pallas as pl
from jax.experimental.pallas import tpu as pltpu
from jax.experimental.pallas import tpu_sc as plsc

N = 10000
E = 320000
D = 128
K = 5

NC, NS, L = 2, 16, 16
NW = NC * NS
EP = E // NW
CH = 80
NCH = EP // CH
NP = 10240
NH = NP // NC
NT = NH + 8
EPS = E // NS
NCS = EPS // CH
RP = NH // NS
ZR = 80
BR = 1024


def _mesh():
    return plsc.VectorSubcoreMesh(
        core_axis_name="c", subcore_axis_name="s",
        num_cores=NC, num_subcores=NS)


def _remap(idx_d, idx_l, lo):
    for w in range(CH // L):
        v = idx_d[pl.ds(w * L, L)] - lo
        oob = (v < 0) | (v >= NH)
        idx_l[pl.ds(w * L, L)] = jnp.where(oob, NH, v)


def _zero_shared(zbuf, sh, sid):
    zv = jnp.zeros((L,), jnp.float32)

    def zb(i, c):
        zbuf[i // 8, pl.ds((i % 8) * L, L)] = zv
        return c
    lax.fori_loop(0, ZR * 8, zb, 0)
    for j in range(RP // ZR):
        pltpu.sync_copy(zbuf, sh.at[pl.ds(sid * RP + j * ZR, ZR)])


def _agg_body(x_hbm, src_hbm, dst_hbm, agg_hbm,
              isrc, idst, il0, il1, r0, r1, zbuf, agg_sh,
              sem0, sem1, scs0, scs1):
    cid = lax.axis_index("c")
    sid = lax.axis_index("s")
    lo = cid * NH

    _zero_shared(zbuf, agg_sh, sid)
    pltpu.sync_copy(src_hbm.at[pl.ds(sid * EPS, EPS)], isrc)
    pltpu.sync_copy(dst_hbm.at[pl.ds(sid * EPS, EPS)], idst)
    plsc.subcore_barrier()

    def fire(c, rows, sem):
        pltpu.async_copy(x_hbm.at[isrc.at[pl.ds(c * CH, CH)]], rows, sem)

    def drain(c, rows, sem):
        pltpu.make_async_copy(x_hbm.at[isrc.at[pl.ds(c * CH, CH)]],
                              rows, sem).wait()

    def drain_sc(rows, il, sem):
        pltpu.make_async_copy(rows, agg_sh.at[il], sem).wait()

    def remap(c, il):
        for w in range(CH // L):
            v = idst[pl.ds(c * CH + w * L, L)] - lo
            oob = (v < 0) | (v >= NH)
            il[pl.ds(w * L, L)] = jnp.where(oob, NH, v)

    fire(0, r0, sem0)

    def pair(i, carry):
        c0 = 2 * i
        c1 = c0 + 1

        @pl.when(i > 0)
        def _():
            drain_sc(r1, il1, scs1)
        fire(c1, r1, sem1)
        drain(c0, r0, sem0)
        remap(c0, il0)
        pltpu.async_copy(r0, agg_sh.at[il0], scs0, add=True)
        drain(c1, r1, sem1)
        remap(c1, il1)

        @pl.when(i < NCS // 2 - 1)
        def _():
            drain_sc(r0, il0, scs0)
            fire(c0 + 2, r0, sem0)
        pltpu.async_copy(r1, agg_sh.at[il1], scs1, add=True)
        return carry
    lax.fori_loop(0, NCS // 2, pair, 0)

    drain_sc(r0, il0, scs0)
    drain_sc(r1, il1, scs1)
    plsc.subcore_barrier()
    pltpu.sync_copy(agg_sh.at[pl.ds(sid * RP, RP)],
                    agg_hbm.at[pl.ds(cid * NH + sid * RP, RP)])


def _agg(X, src, dst):
    f = pl.kernel(
        _agg_body,
        out_type=jax.ShapeDtypeStruct((NP, D), jnp.float32),
        mesh=_mesh(),
        scratch_types=[
            pltpu.VMEM((EPS,), jnp.int32),
            pltpu.VMEM((EPS,), jnp.int32),
            pltpu.VMEM((CH,), jnp.int32),
            pltpu.VMEM((CH,), jnp.int32),
            pltpu.VMEM((CH, D), jnp.float32),
            pltpu.VMEM((CH, D), jnp.float32),
            pltpu.VMEM((ZR, D), jnp.float32),
            pltpu.VMEM_SHARED((NT, D), jnp.float32),
            pltpu.SemaphoreType.DMA,
            pltpu.SemaphoreType.DMA,
            pltpu.SemaphoreType.DMA,
            pltpu.SemaphoreType.DMA,
        ],
    )
    return f(X, src, dst)


def _deg_body(dst_hbm, deg_hbm, idst, il0, il1, ones_b, zbuf, deg_sh,
              scs0, scs1):
    cid = lax.axis_index("c")
    sid = lax.axis_index("s")
    lo = cid * NH

    ov = jnp.ones((L,), jnp.float32)

    def ob(i, c):
        ones_b[i // 8, pl.ds((i % 8) * L, L)] = ov
        return c
    lax.fori_loop(0, CH * 8, ob, 0)

    _zero_shared(zbuf, deg_sh, sid)
    pltpu.sync_copy(dst_hbm.at[pl.ds(sid * EPS, EPS)], idst)
    plsc.subcore_barrier()

    def remap(c, il):
        for w in range(CH // L):
            v = idst[pl.ds(c * CH + w * L, L)] - lo
            oob = (v < 0) | (v >= NH)
            il[pl.ds(w * L, L)] = jnp.where(oob, NH, v)

    def drain_sc(il, sem):
        pltpu.make_async_copy(ones_b, deg_sh.at[il], sem).wait()

    def pair(i, carry):
        c0 = 2 * i
        c1 = c0 + 1

        @pl.when(i > 0)
        def _():
            drain_sc(il0, scs0)
        remap(c0, il0)
        pltpu.async_copy(ones_b, deg_sh.at[il0], scs0, add=True)

        @pl.when(i > 0)
        def _():
            drain_sc(il1, scs1)
        remap(c1, il1)
        pltpu.async_copy(ones_b, deg_sh.at[il1], scs1, add=True)
        return carry
    lax.fori_loop(0, NCS // 2, pair, 0)

    drain_sc(il0, scs0)
    drain_sc(il1, scs1)
    plsc.subcore_barrier()
    pltpu.sync_copy(deg_sh.at[pl.ds(sid * RP, RP)],
                    deg_hbm.at[pl.ds(cid * NH + sid * RP, RP)])


def _deg(dst):
    f = pl.kernel(
        _deg_body,
        out_type=jax.ShapeDtypeStruct((NP, D), jnp.float32),
        mesh=_mesh(),
        scratch_types=[
            pltpu.VMEM((EPS,), jnp.int32),
            pltpu.VMEM((CH,), jnp.int32),
            pltpu.VMEM((CH,), jnp.int32),
            pltpu.VMEM((CH, D), jnp.float32),
            pltpu.VMEM((ZR, D), jnp.float32),
            pltpu.VMEM_SHARED((NT, D), jnp.float32),
            pltpu.SemaphoreType.DMA,
            pltpu.SemaphoreType.DMA,
        ],
    )
    return f(dst)


def _dense_body(z_ref, agg_ref, deg_ref, w_ref, o_ref):
    agg = agg_ref[...]
    deg = deg_ref[:, 0]
    zn = agg / jnp.maximum(deg, 1.0)[:, None]
    h = jnp.concatenate([z_ref[...], zn], axis=1)
    pre = lax.dot_general(h, w_ref[...], (((1,), (1,)), ((), ())),
                          preferred_element_type=jnp.float32)
    s = 1.0 / (1.0 + jnp.exp(-pre))
    o_ref[...] = s * lax.rsqrt(jnp.sum(s * s, axis=1, keepdims=True))


def _dense(Zc, aggp, degp, W):
    return pl.pallas_call(
        _dense_body,
        grid=(NP // BR,),
        in_specs=[
            pl.BlockSpec((BR, D), lambda i: (i, 0)),
            pl.BlockSpec((BR, D), lambda i: (i, 0)),
            pl.BlockSpec((BR, D), lambda i: (i, 0)),
            pl.BlockSpec((D, 2 * D), lambda i: (0, 0)),
        ],
        out_specs=pl.BlockSpec((BR, D), lambda i: (i, 0)),
        out_shape=jax.ShapeDtypeStruct((NP, D), jnp.float32),
    )(Zc, aggp, degp, W)


CHD = 40
NCD = EP // CHD


def _dot_body(z_hbm, src_hbm, dst_hbm, neg_hbm, out_hbm,
              isrc, idst,
              in0, ru0, rv0, rn0, dt0,
              in1, ru1, rv1, rn1, dt1, sem0, sem1, semi0, semi1,
              semo0, semo1):
    cid = lax.axis_index("c")
    sid = lax.axis_index("s")
    wid = cid * NS + sid
    zv = jnp.zeros((L,), jnp.float32)

    bufs = ((in0, ru0, rv0, rn0, dt0, sem0, semi0, semo0),
            (in1, ru1, rv1, rn1, dt1, sem1, semi1, semo1))

    pltpu.sync_copy(src_hbm.at[pl.ds(wid * EP, EP)], isrc)
    pltpu.sync_copy(dst_hbm.at[pl.ds(wid * EP, EP)], idst)

    def zpad(e, c):
        dt0[e, pl.ds(6 * L, L)] = zv
        dt0[e, pl.ds(7 * L, L)] = zv
        dt1[e, pl.ds(6 * L, L)] = zv
        dt1[e, pl.ds(7 * L, L)] = zv
        return c
    lax.fori_loop(0, CHD, zpad, 0)

    def load_idx(c, b):
        off = wid * EP + c * CHD
        pltpu.sync_copy(neg_hbm.at[pl.ds(off * K, CHD * K)], b[0])

    def fire_idx(c, b):
        off = wid * EP + c * CHD
        pltpu.async_copy(neg_hbm.at[pl.ds(off * K, CHD * K)], b[0], b[6])

    def drain_idx(c, b):
        off = wid * EP + c * CHD
        pltpu.make_async_copy(neg_hbm.at[pl.ds(off * K, CHD * K)],
                              b[0], b[6]).wait()

    def fire(c, b):
        inn, ru, rv, rn = b[0], b[1], b[2], b[3]
        sem = b[5]
        pltpu.async_copy(z_hbm.at[isrc.at[pl.ds(c * CHD, CHD)]], ru, sem)
        pltpu.async_copy(z_hbm.at[idst.at[pl.ds(c * CHD, CHD)]], rv, sem)
        for k5 in range(K):
            pltpu.async_copy(z_hbm.at[inn.at[pl.ds(k5 * CHD, CHD)]],
                             rn.at[pl.ds(k5 * CHD, CHD)], sem)

    def drain(c, b):
        inn, ru, rv, rn = b[0], b[1], b[2], b[3]
        sem = b[5]
        pltpu.make_async_copy(z_hbm.at[isrc.at[pl.ds(c * CHD, CHD)]],
                              ru, sem).wait()
        pltpu.make_async_copy(z_hbm.at[idst.at[pl.ds(c * CHD, CHD)]],
                              rv, sem).wait()
        for k5 in range(K):
            pltpu.make_async_copy(z_hbm.at[inn.at[pl.ds(k5 * CHD, CHD)]],
                                  rn.at[pl.ds(k5 * CHD, CHD)], sem).wait()

    def dot16(us, row_ref, r):
        a = us[0] * row_ref[r, pl.ds(0, L)]
        b2 = us[1] * row_ref[r, pl.ds(L, L)]
        for j in range(2, D // L, 2):
            a += us[j] * row_ref[r, pl.ds(j * L, L)]
            b2 += us[j + 1] * row_ref[r, pl.ds((j + 1) * L, L)]
        return a + b2

    def compute(c, b):
        ru, rv, rn, dt = b[1], b[2], b[3], b[4]

        @plsc.parallel_loop(0, CHD, unroll=4)
        def ebody(e):
            us = tuple(ru[e, pl.ds(j * L, L)] for j in range(D // L))
            dt[e, pl.ds(0, L)] = dot16(us, rv, e)
            rb = e * K
            for k in range(K):
                dt[e, pl.ds((1 + k) * L, L)] = dot16(us, rn, rb + k)
        pltpu.async_copy(dt, out_hbm.at[pl.ds(wid * EP + c * CHD, CHD)], b[7])

    def drain_out(c, b):
        pltpu.make_async_copy(b[4], out_hbm.at[pl.ds(wid * EP + c * CHD, CHD)],
                              b[7]).wait()

    load_idx(0, bufs[0])
    fire(0, bufs[0])
    load_idx(1, bufs[1])
    fire(1, bufs[1])

    def pair(i, carry):
        c0 = 2 * i
        c1 = c0 + 1
        last = NCD // 2 - 1
        drain(c0, bufs[0])

        @pl.when(i < last)
        def _():
            fire_idx(c0 + 2, bufs[0])

        @pl.when(i > 0)
        def _():
            drain_out(0, bufs[0])
        compute(c0, bufs[0])

        @pl.when(i < last)
        def _():
            drain_idx(c0 + 2, bufs[0])
            fire(c0 + 2, bufs[0])
        drain(c1, bufs[1])

        @pl.when(i < last)
        def _():
            fire_idx(c1 + 2, bufs[1])

        @pl.when(i > 0)
        def _():
            drain_out(0, bufs[1])
        compute(c1, bufs[1])

        @pl.when(i < last)
        def _():
            drain_idx(c1 + 2, bufs[1])
            fire(c1 + 2, bufs[1])
        return carry
    lax.fori_loop(0, NCD // 2, pair, 0)
    drain_out(0, bufs[0])
    drain_out(0, bufs[1])


def _dot(Z2, src, dst, negf):
    buf_set = [
        pltpu.VMEM((CHD * K,), jnp.int32),
        pltpu.VMEM((CHD, D), jnp.float32),
        pltpu.VMEM((CHD, D), jnp.float32),
        pltpu.VMEM((CHD * K, D), jnp.float32),
        pltpu.VMEM((CHD, 8 * L), jnp.float32),
    ]
    f = pl.kernel(
        _dot_body,
        out_type=jax.ShapeDtypeStruct((E, 8 * L), jnp.float32),
        mesh=_mesh(),
        scratch_types=[pltpu.VMEM((EP,), jnp.int32),
                       pltpu.VMEM((EP,), jnp.int32)]
        + buf_set + buf_set + [pltpu.SemaphoreType.DMA] * 6,
    )
    return f(Z2, src, dst, negf)


def _red_body(d_ref, o_ref):
    i = pl.program_id(0)
    x = d_ref[...]
    j = lax.broadcasted_iota(jnp.int32, (D, 8), 0) // L
    t8 = lax.broadcasted_iota(jnp.int32, (D, 8), 1)
    m = jnp.where(j == t8, jnp.where(t8 == 0, 1.0, -1.0), 0.0)
    s = lax.dot_general(x, m, (((1,), (0,)), ((), ())),
                        preferred_element_type=jnp.float32)

    def ls(v):
        return jnp.minimum(v, 0.0) - jnp.log1p(jnp.exp(-jnp.abs(v)))

    tcol = lax.broadcasted_iota(jnp.int32, s.shape, 1)
    tot = jnp.sum(jnp.where(tcol < 6, ls(s), 0.0))

    @pl.when(i == 0)
    def _():
        o_ref[0, 0] = 0.0
    o_ref[0, 0] += tot


def _reduce(dots_r):
    rows = dots_r.shape[0]
    blk = rows // 40
    return pl.pallas_call(
        _red_body,
        grid=(40,),
        in_specs=[pl.BlockSpec((blk, 128), lambda i: (i, 0))],
        out_specs=pl.BlockSpec((1, 1), lambda i: (0, 0), memory_space=pltpu.SMEM),
        out_shape=jax.ShapeDtypeStruct((1, 1), jnp.float32),
    )(dots_r)


def kernel(Z, W1, W2, edge_index, neg_idx):
    src = edge_index[0]
    dst = edge_index[1]
    negf = neg_idx.reshape(-1)

    Zp = jnp.concatenate([Z, jnp.zeros((NP - N, D), Z.dtype)])
    deg = _deg(dst)
    agg1 = _agg(Zp, src, dst)
    Z1 = _dense(Zp, agg1, deg, W1)
    agg2 = _agg(Z1, src, dst)
    Z2 = _dense(Z1, agg2, deg, W2)

    dots = _dot(Z2, src, dst, negf)
    red = _reduce(dots)
    return -red[0, 0]


# speedup vs baseline: 1.0032x; 1.0032x over previous
import functools

import jax
import jax.numpy as jnp
from jax import lax
from jax.experimental import pallas as pl
from jax.experimental.pallas import tpu as pltpu
from jax.experimental.pallas import tpu_sc as plsc

N = 10000
E = 320000
D = 128
K = 5

NC, NS, L = 2, 16, 16
NW = NC * NS
EP = E // NW
CH = 80
NCH = EP // CH
NP = 10240
NH = NP // NC
NT = NH + 8
EPS = E // NS
NCS = EPS // CH
RP = NH // NS
ZR = 80
BR = 1024


def _mesh():
    return plsc.VectorSubcoreMesh(
        core_axis_name="c", subcore_axis_name="s",
        num_cores=NC, num_subcores=NS)


def _remap(idx_d, idx_l, lo):
    for w in range(CH // L):
        v = idx_d[pl.ds(w * L, L)] - lo
        oob = (v < 0) | (v >= NH)
        idx_l[pl.ds(w * L, L)] = jnp.where(oob, NH, v)


def _zero_shared(zbuf, sh, sid):
    zv = jnp.zeros((L,), jnp.float32)

    def zb(i, c):
        zbuf[i // 8, pl.ds((i % 8) * L, L)] = zv
        return c
    lax.fori_loop(0, ZR * 8, zb, 0)
    for j in range(RP // ZR):
        pltpu.sync_copy(zbuf, sh.at[pl.ds(sid * RP + j * ZR, ZR)])


def _agg_body(x_hbm, src_hbm, dst_hbm, agg_hbm,
              isrc, idst, il0, il1, r0, r1, zbuf, agg_sh,
              sem0, sem1, scs0, scs1):
    cid = lax.axis_index("c")
    sid = lax.axis_index("s")
    lo = cid * NH

    _zero_shared(zbuf, agg_sh, sid)
    pltpu.sync_copy(src_hbm.at[pl.ds(sid * EPS, EPS)], isrc)
    pltpu.sync_copy(dst_hbm.at[pl.ds(sid * EPS, EPS)], idst)
    plsc.subcore_barrier()

    def fire(c, rows, sem):
        pltpu.async_copy(x_hbm.at[isrc.at[pl.ds(c * CH, CH)]], rows, sem)

    def drain(c, rows, sem):
        pltpu.make_async_copy(x_hbm.at[isrc.at[pl.ds(c * CH, CH)]],
                              rows, sem).wait()

    def drain_sc(rows, il, sem):
        pltpu.make_async_copy(rows, agg_sh.at[il], sem).wait()

    def remap(c, il):
        for w in range(CH // L):
            v = idst[pl.ds(c * CH + w * L, L)] - lo
            oob = (v < 0) | (v >= NH)
            il[pl.ds(w * L, L)] = jnp.where(oob, NH, v)

    fire(0, r0, sem0)

    def pair(i, carry):
        c0 = 2 * i
        c1 = c0 + 1

        @pl.when(i > 0)
        def _():
            drain_sc(r1, il1, scs1)
        fire(c1, r1, sem1)
        drain(c0, r0, sem0)
        remap(c0, il0)
        pltpu.async_copy(r0, agg_sh.at[il0], scs0, add=True)
        drain(c1, r1, sem1)
        remap(c1, il1)

        @pl.when(i < NCS // 2 - 1)
        def _():
            drain_sc(r0, il0, scs0)
            fire(c0 + 2, r0, sem0)
        pltpu.async_copy(r1, agg_sh.at[il1], scs1, add=True)
        return carry
    lax.fori_loop(0, NCS // 2, pair, 0)

    drain_sc(r0, il0, scs0)
    drain_sc(r1, il1, scs1)
    plsc.subcore_barrier()
    pltpu.sync_copy(agg_sh.at[pl.ds(sid * RP, RP)],
                    agg_hbm.at[pl.ds(cid * NH + sid * RP, RP)])


def _agg(X, src, dst):
    f = pl.kernel(
        _agg_body,
        out_type=jax.ShapeDtypeStruct((NP, D), jnp.float32),
        mesh=_mesh(),
        scratch_types=[
            pltpu.VMEM((EPS,), jnp.int32),
            pltpu.VMEM((EPS,), jnp.int32),
            pltpu.VMEM((CH,), jnp.int32),
            pltpu.VMEM((CH,), jnp.int32),
            pltpu.VMEM((CH, D), jnp.float32),
            pltpu.VMEM((CH, D), jnp.float32),
            pltpu.VMEM((ZR, D), jnp.float32),
            pltpu.VMEM_SHARED((NT, D), jnp.float32),
            pltpu.SemaphoreType.DMA,
            pltpu.SemaphoreType.DMA,
            pltpu.SemaphoreType.DMA,
            pltpu.SemaphoreType.DMA,
        ],
    )
    return f(X, src, dst)


def _deg_body(dst_hbm, deg_hbm, idst, il0, il1, ones_b, zbuf, deg_sh,
              scs0, scs1):
    cid = lax.axis_index("c")
    sid = lax.axis_index("s")
    lo = cid * NH

    ov = jnp.ones((L,), jnp.float32)

    def ob(i, c):
        ones_b[i // 8, pl.ds((i % 8) * L, L)] = ov
        return c
    lax.fori_loop(0, CH * 8, ob, 0)

    _zero_shared(zbuf, deg_sh, sid)
    pltpu.sync_copy(dst_hbm.at[pl.ds(sid * EPS, EPS)], idst)
    plsc.subcore_barrier()

    def remap(c, il):
        for w in range(CH // L):
            v = idst[pl.ds(c * CH + w * L, L)] - lo
            oob = (v < 0) | (v >= NH)
            il[pl.ds(w * L, L)] = jnp.where(oob, NH, v)

    def drain_sc(il, sem):
        pltpu.make_async_copy(ones_b, deg_sh.at[il], sem).wait()

    def pair(i, carry):
        c0 = 2 * i
        c1 = c0 + 1

        @pl.when(i > 0)
        def _():
            drain_sc(il0, scs0)
        remap(c0, il0)
        pltpu.async_copy(ones_b, deg_sh.at[il0], scs0, add=True)

        @pl.when(i > 0)
        def _():
            drain_sc(il1, scs1)
        remap(c1, il1)
        pltpu.async_copy(ones_b, deg_sh.at[il1], scs1, add=True)
        return carry
    lax.fori_loop(0, NCS // 2, pair, 0)

    drain_sc(il0, scs0)
    drain_sc(il1, scs1)
    plsc.subcore_barrier()
    pltpu.sync_copy(deg_sh.at[pl.ds(sid * RP, RP)],
                    deg_hbm.at[pl.ds(cid * NH + sid * RP, RP)])


def _deg(dst):
    f = pl.kernel(
        _deg_body,
        out_type=jax.ShapeDtypeStruct((NP, D), jnp.float32),
        mesh=_mesh(),
        scratch_types=[
            pltpu.VMEM((EPS,), jnp.int32),
            pltpu.VMEM((CH,), jnp.int32),
            pltpu.VMEM((CH,), jnp.int32),
            pltpu.VMEM((CH, D), jnp.float32),
            pltpu.VMEM((ZR, D), jnp.float32),
            pltpu.VMEM_SHARED((NT, D), jnp.float32),
            pltpu.SemaphoreType.DMA,
            pltpu.SemaphoreType.DMA,
        ],
    )
    return f(dst)


def _dense_body(z_ref, agg_ref, deg_ref, w_ref, o_ref):
    agg = agg_ref[...]
    deg = deg_ref[:, 0]
    zn = agg / jnp.maximum(deg, 1.0)[:, None]
    h = jnp.concatenate([z_ref[...], zn], axis=1)
    pre = lax.dot_general(h, w_ref[...], (((1,), (1,)), ((), ())),
                          preferred_element_type=jnp.float32)
    s = 1.0 / (1.0 + jnp.exp(-pre))
    o_ref[...] = s * lax.rsqrt(jnp.sum(s * s, axis=1, keepdims=True))


def _dense(Zc, aggp, degp, W):
    return pl.pallas_call(
        _dense_body,
        grid=(NP // BR,),
        in_specs=[
            pl.BlockSpec((BR, D), lambda i: (i, 0)),
            pl.BlockSpec((BR, D), lambda i: (i, 0)),
            pl.BlockSpec((BR, D), lambda i: (i, 0)),
            pl.BlockSpec((D, 2 * D), lambda i: (0, 0)),
        ],
        out_specs=pl.BlockSpec((BR, D), lambda i: (i, 0)),
        out_shape=jax.ShapeDtypeStruct((NP, D), jnp.float32),
    )(Zc, aggp, degp, W)


CHD = 40
NCD = EP // CHD


def _dot_body(z_hbm, src_hbm, dst_hbm, neg_hbm, out_hbm,
              isrc, idst,
              in0, ru0, rv0, rn0, dt0,
              in1, ru1, rv1, rn1, dt1, sem0, sem1, semi0, semi1,
              semo0, semo1):
    cid = lax.axis_index("c")
    sid = lax.axis_index("s")
    wid = cid * NS + sid
    zv = jnp.zeros((L,), jnp.float32)

    bufs = ((in0, ru0, rv0, rn0, dt0, sem0, semi0, semo0),
            (in1, ru1, rv1, rn1, dt1, sem1, semi1, semo1))

    pltpu.sync_copy(src_hbm.at[pl.ds(wid * EP, EP)], isrc)
    pltpu.sync_copy(dst_hbm.at[pl.ds(wid * EP, EP)], idst)

    def zpad(e, c):
        dt0[e, pl.ds(6 * L, L)] = zv
        dt0[e, pl.ds(7 * L, L)] = zv
        dt1[e, pl.ds(6 * L, L)] = zv
        dt1[e, pl.ds(7 * L, L)] = zv
        return c
    lax.fori_loop(0, CHD, zpad, 0)

    def load_idx(c, b):
        off = wid * EP + c * CHD
        pltpu.sync_copy(neg_hbm.at[pl.ds(off * K, CHD * K)], b[0])

    def fire_idx(c, b):
        off = wid * EP + c * CHD
        pltpu.async_copy(neg_hbm.at[pl.ds(off * K, CHD * K)], b[0], b[6])

    def drain_idx(c, b):
        off = wid * EP + c * CHD
        pltpu.make_async_copy(neg_hbm.at[pl.ds(off * K, CHD * K)],
                              b[0], b[6]).wait()

    def fire(c, b):
        inn, ru, rv, rn = b[0], b[1], b[2], b[3]
        sem = b[5]
        pltpu.async_copy(z_hbm.at[isrc.at[pl.ds(c * CHD, CHD)]], ru, sem)
        pltpu.async_copy(z_hbm.at[idst.at[pl.ds(c * CHD, CHD)]], rv, sem)
        for k5 in range(K):
            pltpu.async_copy(z_hbm.at[inn.at[pl.ds(k5 * CHD, CHD)]],
                             rn.at[pl.ds(k5 * CHD, CHD)], sem)

    def drain(c, b):
        inn, ru, rv, rn = b[0], b[1], b[2], b[3]
        sem = b[5]
        pltpu.make_async_copy(z_hbm.at[isrc.at[pl.ds(c * CHD, CHD)]],
                              ru, sem).wait()
        pltpu.make_async_copy(z_hbm.at[idst.at[pl.ds(c * CHD, CHD)]],
                              rv, sem).wait()
        for k5 in range(K):
            pltpu.make_async_copy(z_hbm.at[inn.at[pl.ds(k5 * CHD, CHD)]],
                                  rn.at[pl.ds(k5 * CHD, CHD)], sem).wait()

    def dot16(us, row_ref, r):
        a = us[0] * row_ref[r, pl.ds(0, L)]
        b2 = us[1] * row_ref[r, pl.ds(L, L)]
        for j in range(2, D // L, 2):
            a += us[j] * row_ref[r, pl.ds(j * L, L)]
            b2 += us[j + 1] * row_ref[r, pl.ds((j + 1) * L, L)]
        return a + b2

    def compute(c, b):
        ru, rv, rn, dt = b[1], b[2], b[3], b[4]

        @plsc.parallel_loop(0, CHD, unroll=2)
        def ebody(e):
            us = tuple(ru[e, pl.ds(j * L, L)] for j in range(D // L))
            dt[e, pl.ds(0, L)] = dot16(us, rv, e)
            rb = e * K
            for k in range(K):
                dt[e, pl.ds((1 + k) * L, L)] = dot16(us, rn, rb + k)
        pltpu.async_copy(dt, out_hbm.at[pl.ds(wid * EP + c * CHD, CHD)], b[7])

    def drain_out(c, b):
        pltpu.make_async_copy(b[4], out_hbm.at[pl.ds(wid * EP + c * CHD, CHD)],
                              b[7]).wait()

    load_idx(0, bufs[0])
    fire(0, bufs[0])
    load_idx(1, bufs[1])
    fire(1, bufs[1])

    def pair(i, carry):
        c0 = 2 * i
        c1 = c0 + 1
        last = NCD // 2 - 1
        drain(c0, bufs[0])

        @pl.when(i < last)
        def _():
            fire_idx(c0 + 2, bufs[0])

        @pl.when(i > 0)
        def _():
            drain_out(0, bufs[0])
        compute(c0, bufs[0])

        @pl.when(i < last)
        def _():
            drain_idx(c0 + 2, bufs[0])
            fire(c0 + 2, bufs[0])
        drain(c1, bufs[1])

        @pl.when(i < last)
        def _():
            fire_idx(c1 + 2, bufs[1])

        @pl.when(i > 0)
        def _():
            drain_out(0, bufs[1])
        compute(c1, bufs[1])

        @pl.when(i < last)
        def _():
            drain_idx(c1 + 2, bufs[1])
            fire(c1 + 2, bufs[1])
        return carry
    lax.fori_loop(0, NCD // 2, pair, 0)
    drain_out(0, bufs[0])
    drain_out(0, bufs[1])


def _dot(Z2, src, dst, negf):
    buf_set = [
        pltpu.VMEM((CHD * K,), jnp.int32),
        pltpu.VMEM((CHD, D), jnp.float32),
        pltpu.VMEM((CHD, D), jnp.float32),
        pltpu.VMEM((CHD * K, D), jnp.float32),
        pltpu.VMEM((CHD, 8 * L), jnp.float32),
    ]
    f = pl.kernel(
        _dot_body,
        out_type=jax.ShapeDtypeStruct((E, 8 * L), jnp.float32),
        mesh=_mesh(),
        scratch_types=[pltpu.VMEM((EP,), jnp.int32),
                       pltpu.VMEM((EP,), jnp.int32)]
        + buf_set + buf_set + [pltpu.SemaphoreType.DMA] * 6,
    )
    return f(Z2, src, dst, negf)


def _red_body(d_ref, o_ref):
    i = pl.program_id(0)
    x = d_ref[...]
    j = lax.broadcasted_iota(jnp.int32, (D, 8), 0) // L
    t8 = lax.broadcasted_iota(jnp.int32, (D, 8), 1)
    m = jnp.where(j == t8, jnp.where(t8 == 0, 1.0, -1.0), 0.0)
    s = lax.dot_general(x, m, (((1,), (0,)), ((), ())),
                        preferred_element_type=jnp.float32)

    def ls(v):
        return jnp.minimum(v, 0.0) - jnp.log1p(jnp.exp(-jnp.abs(v)))

    tcol = lax.broadcasted_iota(jnp.int32, s.shape, 1)
    tot = jnp.sum(jnp.where(tcol < 6, ls(s), 0.0))

    @pl.when(i == 0)
    def _():
        o_ref[0, 0] = 0.0
    o_ref[0, 0] += tot


def _reduce(dots_r):
    rows = dots_r.shape[0]
    blk = rows // 40
    return pl.pallas_call(
        _red_body,
        grid=(40,),
        in_specs=[pl.BlockSpec((blk, 128), lambda i: (i, 0))],
        out_specs=pl.BlockSpec((1, 1), lambda i: (0, 0), memory_space=pltpu.SMEM),
        out_shape=jax.ShapeDtypeStruct((1, 1), jnp.float32),
    )(dots_r)


def kernel(Z, W1, W2, edge_index, neg_idx):
    src = edge_index[0]
    dst = edge_index[1]
    negf = neg_idx.reshape(-1)

    Zp = jnp.concatenate([Z, jnp.zeros((NP - N, D), Z.dtype)])
    deg = _deg(dst)
    agg1 = _agg(Zp, src, dst)
    Z1 = _dense(Zp, agg1, deg, W1)
    agg2 = _agg(Z1, src, dst)
    Z2 = _dense(Z1, agg2, deg, W2)

    dots = _dot(Z2, src, dst, negf)
    red = _reduce(dots)
    return -red[0, 0]
